# Initial kernel scaffold; baseline (speedup 1.0000x reference)
import jax
import jax.numpy as jnp
from jax.experimental import pallas as pl


def kernel(x, gate_w, w1, w3, w2):
    raise NotImplementedError("write your pallas kernel here")


# speedup vs baseline: 1.0629x; 1.0629x over previous
import functools

import jax
import jax.numpy as jnp
from jax.experimental import pallas as pl
from jax.experimental.pallas import tpu as pltpu

B = 32
D_MODEL = 4096
D_FF = 4096
NUM_EXPERTS = 8
TF = 512
NF = D_FF // TF


def _moe_kernel(x_ref, gw_ref, w1_ref, w3_ref, w2_ref, out_ref,
                dense_w_ref, acc_ref):
    e = pl.program_id(0)
    f = pl.program_id(1)

    @pl.when((e == 0) & (f == 0))
    def _init():
        x = x_ref[:]
        logits = jnp.dot(x, gw_ref[:], preferred_element_type=jnp.float32)
        ecols = jax.lax.broadcasted_iota(jnp.int32, logits.shape, 1)
        m1 = jnp.max(logits, axis=1, keepdims=True)
        i1 = jnp.min(jnp.where(logits == m1, ecols, NUM_EXPERTS),
                     axis=1, keepdims=True)
        masked = jnp.where(ecols == i1, -jnp.inf, logits)
        m2 = jnp.max(masked, axis=1, keepdims=True)
        i2 = jnp.min(jnp.where(masked == m2, ecols, NUM_EXPERTS),
                     axis=1, keepdims=True)
        wa = jax.nn.sigmoid(m1 - m2)
        dense_w_ref[:] = (jnp.where(ecols == i1, wa, 0.0)
                          + jnp.where(ecols == i2, 1.0 - wa, 0.0))
        acc_ref[:] = jnp.zeros_like(acc_ref)

    x = x_ref[:]
    h1 = jnp.dot(x, w1_ref[0], preferred_element_type=jnp.float32)
    h3 = jnp.dot(x, w3_ref[0], preferred_element_type=jnp.float32)
    h = (h1 * jax.nn.sigmoid(h1)) * h3
    ecols = jax.lax.broadcasted_iota(jnp.int32, (B, NUM_EXPERTS), 1)
    scale = jnp.sum(jnp.where(ecols == e, dense_w_ref[:], 0.0),
                    axis=1, keepdims=True)
    acc_ref[:] += jnp.dot(h * scale, w2_ref[0],
                          preferred_element_type=jnp.float32)

    @pl.when((e == NUM_EXPERTS - 1) & (f == NF - 1))
    def _done():
        out_ref[:] = acc_ref[:]


@jax.jit
def kernel(x, gate_w, w1, w3, w2):
    tokens = x.reshape(B, D_MODEL)
    out = pl.pallas_call(
        _moe_kernel,
        grid=(NUM_EXPERTS, NF),
        in_specs=[
            pl.BlockSpec((B, D_MODEL), lambda e, f: (0, 0)),
            pl.BlockSpec((D_MODEL, NUM_EXPERTS), lambda e, f: (0, 0)),
            pl.BlockSpec((1, D_MODEL, TF), lambda e, f: (e, 0, f)),
            pl.BlockSpec((1, D_MODEL, TF), lambda e, f: (e, 0, f)),
            pl.BlockSpec((1, TF, D_MODEL), lambda e, f: (e, f, 0)),
        ],
        out_specs=pl.BlockSpec((B, D_MODEL), lambda e, f: (0, 0)),
        out_shape=jax.ShapeDtypeStruct((B, D_MODEL), jnp.float32),
        scratch_shapes=[
            pltpu.VMEM((B, NUM_EXPERTS), jnp.float32),
            pltpu.VMEM((B, D_MODEL), jnp.float32),
        ],
    )(tokens, gate_w, w1, w3, w2)
    return out.reshape(B, 1, 1, D_MODEL)


# speedup vs baseline: 1.0814x; 1.0174x over previous
import functools

import jax
import jax.numpy as jnp
from jax.experimental import pallas as pl
from jax.experimental.pallas import tpu as pltpu

B = 32
D_MODEL = 4096
D_FF = 4096
NUM_EXPERTS = 8
TF = 256
NF = D_FF // TF


def _moe_kernel(x_ref, gw_ref, w1_ref, w3_ref, w2_ref, out_ref,
                dense_w_ref, acc_ref):
    e = pl.program_id(0)
    f = pl.program_id(1)

    @pl.when((e == 0) & (f == 0))
    def _init():
        x = x_ref[:]
        logits = jnp.dot(x, gw_ref[:], preferred_element_type=jnp.float32)
        ecols = jax.lax.broadcasted_iota(jnp.int32, logits.shape, 1)
        m1 = jnp.max(logits, axis=1, keepdims=True)
        i1 = jnp.min(jnp.where(logits == m1, ecols, NUM_EXPERTS),
                     axis=1, keepdims=True)
        masked = jnp.where(ecols == i1, -jnp.inf, logits)
        m2 = jnp.max(masked, axis=1, keepdims=True)
        i2 = jnp.min(jnp.where(masked == m2, ecols, NUM_EXPERTS),
                     axis=1, keepdims=True)
        wa = jax.nn.sigmoid(m1 - m2)
        dense_w_ref[:] = (jnp.where(ecols == i1, wa, 0.0)
                          + jnp.where(ecols == i2, 1.0 - wa, 0.0))
        acc_ref[:] = jnp.zeros_like(acc_ref)

    x = x_ref[:]
    h1 = jnp.dot(x, w1_ref[0], preferred_element_type=jnp.float32)
    h3 = jnp.dot(x, w3_ref[0], preferred_element_type=jnp.float32)
    h = (h1 * jax.nn.sigmoid(h1)) * h3
    ecols = jax.lax.broadcasted_iota(jnp.int32, (B, NUM_EXPERTS), 1)
    scale = jnp.sum(jnp.where(ecols == e, dense_w_ref[:], 0.0),
                    axis=1, keepdims=True)
    acc_ref[:] += jnp.dot(h * scale, w2_ref[0],
                          preferred_element_type=jnp.float32)

    @pl.when((e == NUM_EXPERTS - 1) & (f == NF - 1))
    def _done():
        out_ref[:] = acc_ref[:]


@jax.jit
def kernel(x, gate_w, w1, w3, w2):
    tokens = x.reshape(B, D_MODEL)
    out = pl.pallas_call(
        _moe_kernel,
        grid=(NUM_EXPERTS, NF),
        in_specs=[
            pl.BlockSpec((B, D_MODEL), lambda e, f: (0, 0)),
            pl.BlockSpec((D_MODEL, NUM_EXPERTS), lambda e, f: (0, 0)),
            pl.BlockSpec((1, D_MODEL, TF), lambda e, f: (e, 0, f)),
            pl.BlockSpec((1, D_MODEL, TF), lambda e, f: (e, 0, f)),
            pl.BlockSpec((1, TF, D_MODEL), lambda e, f: (e, f, 0)),
        ],
        out_specs=pl.BlockSpec((B, D_MODEL), lambda e, f: (0, 0)),
        out_shape=jax.ShapeDtypeStruct((B, D_MODEL), jnp.float32),
        scratch_shapes=[
            pltpu.VMEM((B, NUM_EXPERTS), jnp.float32),
            pltpu.VMEM((B, D_MODEL), jnp.float32),
        ],
    )(tokens, gate_w, w1, w3, w2)
    return out.reshape(B, 1, 1, D_MODEL)
